# baseline (speedup 1.0000x reference)
import jax
import jax.numpy as jnp
import numpy as np
from jax.experimental import pallas as pl
from jax.experimental.pallas import tpu as pltpu

N_CHANNELS = 3
BLOCK_N = 2048

_LOG2E = float(np.log2(np.e))
_KQ = -0.5 * _LOG2E
_C2 = float(np.log2(_LOG2E / 2.0))
_COS_COEF = (1.0 - 1.25e-6, -19.739202, 64.93908, -85.4497, 60.16561,
             -25.964163, 6.5281506)


def _coef_body(pt_ref, colt_ref, cf_ref, cs_ref):
    mx = pt_ref[:, 0:1]
    my = pt_ref[:, 1:2]
    m00 = pt_ref[:, 2:3]
    m01 = pt_ref[:, 3:4]
    m10 = pt_ref[:, 4:5]
    m11 = pt_ref[:, 5:6]
    rot = pt_ref[:, 6:7]
    freq = pt_ref[:, 7:8]
    off = pt_ref[:, 8:9]
    ftp = pt_ref[:, 9:10]
    logw = pt_ref[:, 10:11]

    kq = jnp.float32(_KQ)
    d0 = -(m00 * mx + m01 * my)
    d1 = -(m10 * mx + m11 * my)
    qa = kq * (m00 * m00 + m10 * m10)
    qb = (2.0 * kq) * (m00 * m01 + m10 * m11)
    qc = kq * (m01 * m01 + m11 * m11)
    qd = (2.0 * kq) * (m00 * d0 + m10 * d1)
    qe = (2.0 * kq) * (m01 * d0 + m11 * d1)

    c = jnp.cos(rot)
    s = jnp.sin(rot)
    f2 = 2.0 * freq
    fa = f2 * c
    fb = f2 * s
    fc = off * (1.0 / np.pi) - (fa * mx + fb * my)

    p = jnp.exp(ftp)
    kw = jnp.exp2(_C2 / p)
    p2 = p * jnp.float32(_LOG2E)
    hiw = (0.5 * kw) * jnp.exp(-2.0 * logw)
    hw_eps = hiw + 1e-12

    zero = jnp.zeros_like(mx)
    cf_ref[:, :] = jnp.concatenate(
        [qa, qb, qc, qd, qe, fa, fb, fc, hiw, hw_eps, p2,
         zero, zero, zero, zero, zero], axis=1)

    zeta = kq * (d0 * d0 + d1 * d1)
    cs_ref[:, :] = colt_ref[:, :] * jnp.exp2(zeta)


def _main_body(xt_ref, cf_ref, cs_ref, out_ref):
    qa = cf_ref[:, 0:1]
    qb = cf_ref[:, 1:2]
    qc = cf_ref[:, 2:3]
    qd = cf_ref[:, 3:4]
    qe = cf_ref[:, 4:5]
    fa = cf_ref[:, 5:6]
    fb = cf_ref[:, 6:7]
    fc = cf_ref[:, 7:8]
    hiw = cf_ref[:, 8:9]
    hw_eps = cf_ref[:, 9:10]
    p2 = cf_ref[:, 10:11]

    x0 = xt_ref[0:1, :]
    x1 = xt_ref[1:2, :]

    nusq = (qa * x0 + (qb * x1 + qd)) * x0 + (qc * x1 + qe) * x1
    v = fa * x0 + (fb * x1 + fc)

    r = jax.lax.round(v, jax.lax.RoundingMethod.TO_NEAREST_EVEN)
    sf = v - r
    t = sf * sf
    ct = jnp.float32(_COS_COEF[6])
    for k in (5, 4, 3, 2, 1, 0):
        ct = ct * t + jnp.float32(_COS_COEF[k])
    base = hw_eps - ct * hiw
    e1 = jnp.exp2(p2 * jnp.log(base))
    vals = jnp.exp2(nusq - e1)

    out_ref[:, :] = jax.lax.dot_general(
        cs_ref[:, :], vals, (((0,), (0,)), ((), ())),
        preferred_element_type=jnp.float32)


@jax.jit
def kernel(x, gaussian_means, gaussian_mats, subgaussian_frequency,
           subgaussian_offset, subgaussian_flat_top_power,
           subgaussian_width, subgaussian_rotation, colors):
    n_pix = x.shape[0]
    w = gaussian_means.shape[0]

    params_t = jnp.concatenate([
        gaussian_means,
        gaussian_mats.reshape(w, 4),
        subgaussian_rotation,
        subgaussian_frequency,
        subgaussian_offset,
        subgaussian_flat_top_power,
        subgaussian_width,
        jnp.zeros((w, 5), jnp.float32),
    ], axis=1)
    xt = x.T

    cf, cs = pl.pallas_call(
        _coef_body,
        in_specs=[
            pl.BlockSpec((w, 16), lambda: (0, 0)),
            pl.BlockSpec((w, N_CHANNELS), lambda: (0, 0)),
        ],
        out_specs=[
            pl.BlockSpec((w, 16), lambda: (0, 0)),
            pl.BlockSpec((w, N_CHANNELS), lambda: (0, 0)),
        ],
        out_shape=[
            jax.ShapeDtypeStruct((w, 16), jnp.float32),
            jax.ShapeDtypeStruct((w, N_CHANNELS), jnp.float32),
        ],
    )(params_t, colors)

    out_t = pl.pallas_call(
        _main_body,
        grid=(n_pix // BLOCK_N,),
        in_specs=[
            pl.BlockSpec((2, BLOCK_N), lambda i: (0, i)),
            pl.BlockSpec((w, 16), lambda i: (0, 0)),
            pl.BlockSpec((w, N_CHANNELS), lambda i: (0, 0)),
        ],
        out_specs=pl.BlockSpec((N_CHANNELS, BLOCK_N), lambda i: (0, i)),
        out_shape=jax.ShapeDtypeStruct((N_CHANNELS, n_pix), jnp.float32),
        compiler_params=pltpu.CompilerParams(
            dimension_semantics=("parallel",),
        ),
    )(xt, cf, cs)
    return out_t.T


# speedup vs baseline: 1.1291x; 1.1291x over previous
import jax
import jax.numpy as jnp
import numpy as np
from jax.experimental import pallas as pl
from jax.experimental.pallas import tpu as pltpu

N_CHANNELS = 3
BLOCK_N = 4096

_LOG2E = float(np.log2(np.e))
_KQ = -0.5 * _LOG2E
_C2 = float(np.log2(_LOG2E / 2.0))
_COS_COEF = (0.9999983 - 2.5e-6, -19.738913, 64.92748, -85.26424, 58.774673,
             -21.06805)


def _coef_body(pt_ref, colt_ref, cf_ref, cs_ref):
    mx = pt_ref[:, 0:1]
    my = pt_ref[:, 1:2]
    m00 = pt_ref[:, 2:3]
    m01 = pt_ref[:, 3:4]
    m10 = pt_ref[:, 4:5]
    m11 = pt_ref[:, 5:6]
    rot = pt_ref[:, 6:7]
    freq = pt_ref[:, 7:8]
    off = pt_ref[:, 8:9]
    ftp = pt_ref[:, 9:10]
    logw = pt_ref[:, 10:11]

    kq = jnp.float32(_KQ)
    d0 = -(m00 * mx + m01 * my)
    d1 = -(m10 * mx + m11 * my)
    qa = kq * (m00 * m00 + m10 * m10)
    qb = (2.0 * kq) * (m00 * m01 + m10 * m11)
    qc = kq * (m01 * m01 + m11 * m11)
    qd = (2.0 * kq) * (m00 * d0 + m10 * d1)
    qe = (2.0 * kq) * (m01 * d0 + m11 * d1)

    c = jnp.cos(rot)
    s = jnp.sin(rot)
    f2 = 2.0 * freq
    fa = f2 * c
    fb = f2 * s
    fc = off * (1.0 / np.pi) - (fa * mx + fb * my)

    p = jnp.exp(ftp)
    kw = jnp.exp2(_C2 / p)
    p2 = p * jnp.float32(_LOG2E)
    hiw = (0.5 * kw) * jnp.exp(-2.0 * logw)
    hw_eps = hiw + 1e-12

    zero = jnp.zeros_like(mx)
    cf_ref[:, :] = jnp.concatenate(
        [qa, qb, qc, qd, qe, fa, fb, fc, hiw, hw_eps, p2,
         zero, zero, zero, zero, zero], axis=1)

    zeta = kq * (d0 * d0 + d1 * d1)
    cs_ref[:, :] = colt_ref[:, :] * jnp.exp2(zeta)


def _main_body(xt_ref, cf_ref, cs_ref, out_ref):
    qa = cf_ref[:, 0:1]
    qb = cf_ref[:, 1:2]
    qc = cf_ref[:, 2:3]
    qd = cf_ref[:, 3:4]
    qe = cf_ref[:, 4:5]
    fa = cf_ref[:, 5:6]
    fb = cf_ref[:, 6:7]
    fc = cf_ref[:, 7:8]
    hiw = cf_ref[:, 8:9]
    hw_eps = cf_ref[:, 9:10]
    p2 = cf_ref[:, 10:11]

    x0 = xt_ref[0:1, :]
    x1 = xt_ref[1:2, :]

    nusq = (qa * x0 + (qb * x1 + qd)) * x0 + (qc * x1 + qe) * x1
    v = fa * x0 + (fb * x1 + fc)

    r = jax.lax.round(v, jax.lax.RoundingMethod.TO_NEAREST_EVEN)
    sf = v - r
    t = sf * sf
    ct = jnp.float32(_COS_COEF[5])
    for k in (4, 3, 2, 1, 0):
        ct = ct * t + jnp.float32(_COS_COEF[k])
    base = hw_eps - ct * hiw
    e1 = jnp.exp2(p2 * jnp.log(base))
    vals = jnp.exp2(nusq - e1)

    out_ref[:, :] = jax.lax.dot_general(
        cs_ref[:, :], vals, (((0,), (0,)), ((), ())),
        preferred_element_type=jnp.float32)


@jax.jit
def kernel(x, gaussian_means, gaussian_mats, subgaussian_frequency,
           subgaussian_offset, subgaussian_flat_top_power,
           subgaussian_width, subgaussian_rotation, colors):
    n_pix = x.shape[0]
    w = gaussian_means.shape[0]

    params_t = jnp.concatenate([
        gaussian_means,
        gaussian_mats.reshape(w, 4),
        subgaussian_rotation,
        subgaussian_frequency,
        subgaussian_offset,
        subgaussian_flat_top_power,
        subgaussian_width,
        jnp.zeros((w, 5), jnp.float32),
    ], axis=1)
    xt = x.T

    cf, cs = pl.pallas_call(
        _coef_body,
        in_specs=[
            pl.BlockSpec((w, 16), lambda: (0, 0)),
            pl.BlockSpec((w, N_CHANNELS), lambda: (0, 0)),
        ],
        out_specs=[
            pl.BlockSpec((w, 16), lambda: (0, 0)),
            pl.BlockSpec((w, N_CHANNELS), lambda: (0, 0)),
        ],
        out_shape=[
            jax.ShapeDtypeStruct((w, 16), jnp.float32),
            jax.ShapeDtypeStruct((w, N_CHANNELS), jnp.float32),
        ],
    )(params_t, colors)

    out_t = pl.pallas_call(
        _main_body,
        grid=(n_pix // BLOCK_N,),
        in_specs=[
            pl.BlockSpec((2, BLOCK_N), lambda i: (0, i)),
            pl.BlockSpec((w, 16), lambda i: (0, 0)),
            pl.BlockSpec((w, N_CHANNELS), lambda i: (0, 0)),
        ],
        out_specs=pl.BlockSpec((N_CHANNELS, BLOCK_N), lambda i: (0, i)),
        out_shape=jax.ShapeDtypeStruct((N_CHANNELS, n_pix), jnp.float32),
        compiler_params=pltpu.CompilerParams(
            dimension_semantics=("parallel",),
        ),
    )(xt, cf, cs)
    return out_t.T
